# Initial kernel scaffold; baseline (speedup 1.0000x reference)
import jax
import jax.numpy as jnp
from jax.experimental import pallas as pl


def kernel(x, lps):
    raise NotImplementedError("write your pallas kernel here")


# speedup vs baseline: 3.3492x; 3.3492x over previous
import functools

import jax
import jax.numpy as jnp
import numpy as np
from jax import lax
from jax.experimental import pallas as pl
from jax.experimental.pallas import tpu as pltpu
from jax.experimental.pallas import tpu_sc as plsc

_NUM_LP = 128
_DIM = 1024
_INPUT_LEN = 4096
_BATCH = 4
_TOTAL = _INPUT_LEN + _NUM_LP

_NC = 2
_NS = 16
_NW = _NC * _NS

_XROWS = _BATCH * _INPUT_LEN
_X_PER_W = _XROWS // _NW
_CHUNK = 64
_NCHUNK = _X_PER_W // _CHUNK
_LP_TOTAL = _BATCH * _NUM_LP
_LP_PER_W = _LP_TOTAL // _NW


def _static_indices():
    insert_idxs = np.linspace(0, _TOTAL - 1, _NUM_LP).astype(np.int64)
    perm = np.full((_TOTAL,), -1, dtype=np.int64)
    perm[insert_idxs] = np.arange(_INPUT_LEN, _INPUT_LEN + _NUM_LP)
    perm[perm == -1] = np.arange(_INPUT_LEN)
    inv = np.empty((_TOTAL,), dtype=np.int64)
    inv[perm] = np.arange(_TOTAL)
    b_off = (np.arange(_BATCH) * _TOTAL)[:, None]
    x_dst = (b_off + inv[:_INPUT_LEN][None, :]).reshape(-1).astype(np.int32)
    lp_dst = (b_off + inv[_INPUT_LEN:][None, :]).reshape(-1).astype(np.int32)
    lp_src = np.tile(np.arange(_NUM_LP, dtype=np.int32), _BATCH)
    return x_dst, lp_src, lp_dst


_X_DST, _LP_SRC, _LP_DST = _static_indices()


def _body(x_hbm, lps_hbm, xdst_hbm, lpsrc_hbm, lpdst_hbm, out_hbm,
          idx_v, buf, lpsrc_v, lpdst_v, lpbuf, sem):
    wid = lax.axis_index("s") * _NC + lax.axis_index("c")

    lbase = wid * _LP_PER_W
    pltpu.sync_copy(lpsrc_hbm.at[pl.ds(lbase, _LP_PER_W)], lpsrc_v)
    pltpu.sync_copy(lpdst_hbm.at[pl.ds(lbase, _LP_PER_W)], lpdst_v)
    pltpu.async_copy(lps_hbm.at[lpsrc_v], lpbuf, sem).wait()
    pltpu.async_copy(lpbuf, out_hbm.at[lpdst_v], sem).wait()

    base = wid * _X_PER_W

    def chunk(i, carry):
        start = base + i * _CHUNK
        pltpu.sync_copy(xdst_hbm.at[pl.ds(start, _CHUNK)], idx_v)
        pltpu.sync_copy(x_hbm.at[pl.ds(start, _CHUNK)], buf)
        pltpu.async_copy(buf, out_hbm.at[idx_v], sem).wait()
        return carry

    lax.fori_loop(0, _NCHUNK, chunk, 0)


@jax.jit
def _interleave(x2d, lps, xdst, lpsrc, lpdst):
    mesh = plsc.VectorSubcoreMesh(
        core_axis_name="c", subcore_axis_name="s",
        num_cores=_NC, num_subcores=_NS)
    return pl.kernel(
        _body,
        out_type=jax.ShapeDtypeStruct((_BATCH * _TOTAL, _DIM), jnp.float32),
        mesh=mesh,
        scratch_types=[
            pltpu.VMEM((_CHUNK,), jnp.int32),
            pltpu.VMEM((_CHUNK, _DIM), jnp.float32),
            pltpu.VMEM((_LP_PER_W,), jnp.int32),
            pltpu.VMEM((_LP_PER_W,), jnp.int32),
            pltpu.VMEM((_LP_PER_W, _DIM), jnp.float32),
            pltpu.SemaphoreType.DMA,
        ],
    )(x2d, lps, xdst, lpsrc, lpdst)


def kernel(x, lps):
    x2d = x.reshape(_XROWS, _DIM)
    out = _interleave(x2d, lps,
                      jnp.asarray(_X_DST), jnp.asarray(_LP_SRC),
                      jnp.asarray(_LP_DST))
    return out.reshape(_BATCH, _TOTAL, _DIM)


# speedup vs baseline: 3.4291x; 1.0238x over previous
import jax
import jax.numpy as jnp
import numpy as np
from jax import lax
from jax.experimental import pallas as pl
from jax.experimental.pallas import tpu as pltpu
from jax.experimental.pallas import tpu_sc as plsc

_NUM_LP = 128
_DIM = 1024
_INPUT_LEN = 4096
_BATCH = 4
_TOTAL = _INPUT_LEN + _NUM_LP

_NC = 2
_NS = 16
_NW = _NC * _NS

_XROWS = _BATCH * _INPUT_LEN
_X_PER_W = _XROWS // _NW
_CHUNK = 32
_NCHUNK = _X_PER_W // _CHUNK
_NBUF = 3
_LP_TOTAL = _BATCH * _NUM_LP
_LP_PER_W = _LP_TOTAL // _NW


def _static_indices():
    insert_idxs = np.linspace(0, _TOTAL - 1, _NUM_LP).astype(np.int64)
    perm = np.full((_TOTAL,), -1, dtype=np.int64)
    perm[insert_idxs] = np.arange(_INPUT_LEN, _INPUT_LEN + _NUM_LP)
    perm[perm == -1] = np.arange(_INPUT_LEN)
    inv = np.empty((_TOTAL,), dtype=np.int64)
    inv[perm] = np.arange(_TOTAL)
    b_off = (np.arange(_BATCH) * _TOTAL)[:, None]
    x_dst = (b_off + inv[:_INPUT_LEN][None, :]).reshape(
        _NW, _NCHUNK, _CHUNK).astype(np.int32)
    lp_dst = (b_off + inv[_INPUT_LEN:][None, :]).reshape(-1).astype(np.int32)
    lp_src = np.tile(np.arange(_NUM_LP, dtype=np.int32), _BATCH)
    return x_dst, lp_src, lp_dst


_X_DST, _LP_SRC, _LP_DST = _static_indices()


def _body(x_hbm, lps_hbm, xdst_hbm, lpsrc_hbm, lpdst_hbm, out_hbm,
          idx_v, b0, b1, b2, lpsrc_v, lpdst_v, lpbuf,
          r0, r1, r2, w0, w1, w2, lpsem):
    wid = lax.axis_index("s") * _NC + lax.axis_index("c")
    bufs = (b0, b1, b2)
    rsems = (r0, r1, r2)
    wsems = (w0, w1, w2)

    lbase = wid * _LP_PER_W
    pltpu.sync_copy(lpsrc_hbm.at[pl.ds(lbase, _LP_PER_W)], lpsrc_v)
    pltpu.sync_copy(lpdst_hbm.at[pl.ds(lbase, _LP_PER_W)], lpdst_v)
    lp_gather = pltpu.async_copy(lps_hbm.at[lpsrc_v], lpbuf, lpsem)

    pltpu.sync_copy(xdst_hbm.at[wid], idx_v)

    base = wid * _X_PER_W

    def read_start(i):
        return pltpu.async_copy(
            x_hbm.at[pl.ds(base + i * _CHUNK, _CHUNK)],
            bufs[i % _NBUF], rsems[i % _NBUF])

    writes = {}
    reads = {0: read_start(0)}
    for i in range(_NCHUNK):
        reads.pop(i).wait()
        writes[i] = pltpu.async_copy(
            bufs[i % _NBUF], out_hbm.at[idx_v.at[i]], wsems[i % _NBUF])
        if i + 1 < _NCHUNK:
            if i + 1 - _NBUF >= 0:
                writes.pop(i + 1 - _NBUF).wait()
            reads[i + 1] = read_start(i + 1)
    for i in sorted(writes):
        writes.pop(i).wait()

    lp_gather.wait()
    pltpu.async_copy(lpbuf, out_hbm.at[lpdst_v], lpsem).wait()


@jax.jit
def _interleave(x2d, lps, xdst, lpsrc, lpdst):
    mesh = plsc.VectorSubcoreMesh(
        core_axis_name="c", subcore_axis_name="s",
        num_cores=_NC, num_subcores=_NS)
    return pl.kernel(
        _body,
        out_type=jax.ShapeDtypeStruct((_BATCH * _TOTAL, _DIM), jnp.float32),
        mesh=mesh,
        scratch_types=[
            pltpu.VMEM((_NCHUNK, _CHUNK), jnp.int32),
            pltpu.VMEM((_CHUNK, _DIM), jnp.float32),
            pltpu.VMEM((_CHUNK, _DIM), jnp.float32),
            pltpu.VMEM((_CHUNK, _DIM), jnp.float32),
            pltpu.VMEM((_LP_PER_W,), jnp.int32),
            pltpu.VMEM((_LP_PER_W,), jnp.int32),
            pltpu.VMEM((_LP_PER_W, _DIM), jnp.float32),
            pltpu.SemaphoreType.DMA,
            pltpu.SemaphoreType.DMA,
            pltpu.SemaphoreType.DMA,
            pltpu.SemaphoreType.DMA,
            pltpu.SemaphoreType.DMA,
            pltpu.SemaphoreType.DMA,
            pltpu.SemaphoreType.DMA,
        ],
    )(x2d, lps, xdst, lpsrc, lpdst)


def kernel(x, lps):
    x2d = x.reshape(_XROWS, _DIM)
    out = _interleave(x2d, lps,
                      jnp.asarray(_X_DST), jnp.asarray(_LP_SRC),
                      jnp.asarray(_LP_DST))
    return out.reshape(_BATCH, _TOTAL, _DIM)


# speedup vs baseline: 3.4335x; 1.0013x over previous
import jax
import jax.numpy as jnp
import numpy as np
from jax import lax
from jax.experimental import pallas as pl
from jax.experimental.pallas import tpu as pltpu
from jax.experimental.pallas import tpu_sc as plsc

_NUM_LP = 128
_DIM = 1024
_INPUT_LEN = 4096
_BATCH = 4
_TOTAL = _INPUT_LEN + _NUM_LP

_NC = 2
_NS = 16
_NW = _NC * _NS

_XROWS = _BATCH * _INPUT_LEN
_X_PER_W = _XROWS // _NW
_CHUNK = 32
_NCHUNK = _X_PER_W // _CHUNK
_NBUF = 3
_LP_TOTAL = _BATCH * _NUM_LP
_LP_PER_W = _LP_TOTAL // _NW


def _static_indices():
    insert_idxs = np.linspace(0, _TOTAL - 1, _NUM_LP).astype(np.int64)
    perm = np.full((_TOTAL,), -1, dtype=np.int64)
    perm[insert_idxs] = np.arange(_INPUT_LEN, _INPUT_LEN + _NUM_LP)
    perm[perm == -1] = np.arange(_INPUT_LEN)
    inv = np.empty((_TOTAL,), dtype=np.int64)
    inv[perm] = np.arange(_TOTAL)
    b_off = (np.arange(_BATCH) * _TOTAL)[:, None]
    x_dst = (b_off + inv[:_INPUT_LEN][None, :]).reshape(
        _NW, _NCHUNK, _CHUNK).astype(np.int32)
    lp_dst = (b_off + inv[_INPUT_LEN:][None, :]).reshape(-1).astype(np.int32)
    lp_src = np.tile(np.arange(_NUM_LP, dtype=np.int32), _BATCH)
    return x_dst, lp_src, lp_dst


_X_DST, _LP_SRC, _LP_DST = _static_indices()


def _body(x_hbm, lps_hbm, xdst_hbm, lpsrc_hbm, lpdst_hbm, out_hbm,
          idx_v, b0, b1, b2, lpsrc_v, lpdst_v, lpbuf,
          r0, r1, r2, w0, w1, w2, lpsem):
    wid = lax.axis_index("s") * _NC + lax.axis_index("c")
    bufs = (b0, b1, b2)
    rsems = (r0, r1, r2)
    wsems = (w0, w1, w2)

    lbase = wid * _LP_PER_W
    pltpu.sync_copy(lpsrc_hbm.at[pl.ds(lbase, _LP_PER_W)], lpsrc_v)
    pltpu.sync_copy(lpdst_hbm.at[pl.ds(lbase, _LP_PER_W)], lpdst_v)
    lp_gather = pltpu.async_copy(lps_hbm.at[lpsrc_v], lpbuf, lpsem)

    pltpu.sync_copy(xdst_hbm.at[wid], idx_v)

    base = wid * _X_PER_W

    def read_start(i):
        return pltpu.async_copy(
            x_hbm.at[pl.ds(base + i * _CHUNK, _CHUNK)],
            bufs[i % _NBUF], rsems[i % _NBUF])

    writes = {}
    reads = {0: read_start(0)}
    for i in range(_NCHUNK):
        reads.pop(i).wait()
        writes[i] = pltpu.async_copy(
            bufs[i % _NBUF], out_hbm.at[idx_v.at[i]], wsems[i % _NBUF])
        if i + 1 < _NCHUNK:
            if i + 1 - _NBUF >= 0:
                writes.pop(i + 1 - _NBUF).wait()
            reads[i + 1] = read_start(i + 1)
    for i in sorted(writes):
        writes.pop(i).wait()

    lp_gather.wait()
    pltpu.async_copy(lpbuf, out_hbm.at[lpdst_v], lpsem).wait()


@jax.jit
def _run(x, lps):
    x2d = x.reshape(_XROWS, _DIM)
    mesh = plsc.VectorSubcoreMesh(
        core_axis_name="c", subcore_axis_name="s",
        num_cores=_NC, num_subcores=_NS)
    out = pl.kernel(
        _body,
        out_type=jax.ShapeDtypeStruct((_BATCH * _TOTAL, _DIM), jnp.float32),
        mesh=mesh,
        scratch_types=[
            pltpu.VMEM((_NCHUNK, _CHUNK), jnp.int32),
            pltpu.VMEM((_CHUNK, _DIM), jnp.float32),
            pltpu.VMEM((_CHUNK, _DIM), jnp.float32),
            pltpu.VMEM((_CHUNK, _DIM), jnp.float32),
            pltpu.VMEM((_LP_PER_W,), jnp.int32),
            pltpu.VMEM((_LP_PER_W,), jnp.int32),
            pltpu.VMEM((_LP_PER_W, _DIM), jnp.float32),
            pltpu.SemaphoreType.DMA,
            pltpu.SemaphoreType.DMA,
            pltpu.SemaphoreType.DMA,
            pltpu.SemaphoreType.DMA,
            pltpu.SemaphoreType.DMA,
            pltpu.SemaphoreType.DMA,
            pltpu.SemaphoreType.DMA,
        ],
    )(x2d, lps, jnp.asarray(_X_DST), jnp.asarray(_LP_SRC),
      jnp.asarray(_LP_DST))
    return out.reshape(_BATCH, _TOTAL, _DIM)


def kernel(x, lps):
    return _run(x, lps)


# speedup vs baseline: 3.6323x; 1.0579x over previous
import jax
import jax.numpy as jnp
from jax import lax
from jax.experimental import pallas as pl
from jax.experimental.pallas import tpu as pltpu
from jax.experimental.pallas import tpu_sc as plsc

_NUM_LP = 128
_DIM = 1024
_INPUT_LEN = 4096
_BATCH = 4
_TOTAL = _INPUT_LEN + _NUM_LP

_NC = 2
_NS = 16
_NW = _NC * _NS

_XROWS = _BATCH * _INPUT_LEN
_X_PER_W = _XROWS // _NW
_CHUNK = 32
_NCHUNK = _X_PER_W // _CHUNK
_NBUF = 3
_LP_PER_W = _BATCH * _NUM_LP // _NW


def _body(x_hbm, lps_hbm, out_hbm,
          idx_v, b0, b1, b2, lpsrc_v, lpdst_v, lpbuf,
          r0, r1, r2, w0, w1, w2, lpsem):
    wid = lax.axis_index("s") * _NC + lax.axis_index("c")
    base = wid * _X_PER_W
    bufs = (b0, b1, b2)
    rsems = (r0, r1, r2)
    wsems = (w0, w1, w2)
    lane = lax.iota(jnp.int32, 16)

    g = wid * _LP_PER_W + lane
    k = g & (_NUM_LP - 1)
    lpsrc_v[...] = k
    lpdst_v[...] = (g >> 7) * _TOTAL + 33 * k + ((32 * k * 4129) >> 19)
    lp_gather = pltpu.async_copy(lps_hbm.at[lpsrc_v], lpbuf, lpsem)

    def read_start(i):
        return pltpu.async_copy(
            x_hbm.at[pl.ds(base + i * _CHUNK, _CHUNK)],
            bufs[i % _NBUF], rsems[i % _NBUF])

    reads = {0: read_start(0)}

    for c in range(_NCHUNK):
        for j in range(_CHUNK // 16):
            i = base + c * _CHUNK + j * 16 + lane
            r = i & (_INPUT_LEN - 1)
            idx_v[c, pl.ds(j * 16, 16)] = (
                i + ((i >> 12) << 7) + ((127 * r + 126) >> 12) + 1)

    lp_scatter = None
    writes = {}
    for i in range(_NCHUNK):
        reads.pop(i).wait()
        writes[i] = pltpu.async_copy(
            bufs[i % _NBUF], out_hbm.at[idx_v.at[i]], wsems[i % _NBUF])
        if i == 0:
            lp_gather.wait()
            lp_scatter = pltpu.async_copy(lpbuf, out_hbm.at[lpdst_v], lpsem)
        if i + 1 < _NCHUNK:
            if i + 1 - _NBUF >= 0:
                writes.pop(i + 1 - _NBUF).wait()
            reads[i + 1] = read_start(i + 1)
    for i in sorted(writes):
        writes.pop(i).wait()
    lp_scatter.wait()


@jax.jit
def _run(x, lps):
    x2d = x.reshape(_XROWS, _DIM)
    mesh = plsc.VectorSubcoreMesh(
        core_axis_name="c", subcore_axis_name="s",
        num_cores=_NC, num_subcores=_NS)
    out = pl.kernel(
        _body,
        out_type=jax.ShapeDtypeStruct((_BATCH * _TOTAL, _DIM), jnp.float32),
        mesh=mesh,
        scratch_types=[
            pltpu.VMEM((_NCHUNK, _CHUNK), jnp.int32),
            pltpu.VMEM((_CHUNK, _DIM), jnp.float32),
            pltpu.VMEM((_CHUNK, _DIM), jnp.float32),
            pltpu.VMEM((_CHUNK, _DIM), jnp.float32),
            pltpu.VMEM((16,), jnp.int32),
            pltpu.VMEM((16,), jnp.int32),
            pltpu.VMEM((_LP_PER_W, _DIM), jnp.float32),
            pltpu.SemaphoreType.DMA,
            pltpu.SemaphoreType.DMA,
            pltpu.SemaphoreType.DMA,
            pltpu.SemaphoreType.DMA,
            pltpu.SemaphoreType.DMA,
            pltpu.SemaphoreType.DMA,
            pltpu.SemaphoreType.DMA,
        ],
    )(x2d, lps)
    return out.reshape(_BATCH, _TOTAL, _DIM)


def kernel(x, lps):
    return _run(x, lps)


# speedup vs baseline: 3.7592x; 1.0349x over previous
import jax
import jax.numpy as jnp
from jax import lax
from jax.experimental import pallas as pl
from jax.experimental.pallas import tpu as pltpu
from jax.experimental.pallas import tpu_sc as plsc

_NUM_LP = 128
_DIM = 1024
_INPUT_LEN = 4096
_BATCH = 4
_TOTAL = _INPUT_LEN + _NUM_LP

_NC = 2
_NS = 16
_NW = _NC * _NS

_XROWS = _BATCH * _INPUT_LEN
_X_PER_W = _XROWS // _NW
_CHUNK = 32
_NCHUNK = _X_PER_W // _CHUNK
_NBUF = 2
_LP_PER_W = _BATCH * _NUM_LP // _NW


def _body(x_hbm, lps_hbm, out_hbm,
          idx_v, b0, b1, lpsrc_v, lpdst_v, lpbuf,
          r0, r1, w0, w1, lpsem):
    wid = lax.axis_index("s") * _NC + lax.axis_index("c")
    base = wid * _X_PER_W
    bufs = (b0, b1)
    rsems = (r0, r1)
    wsems = (w0, w1)
    lane = lax.iota(jnp.int32, 16)

    g = wid * _LP_PER_W + lane
    k = g & (_NUM_LP - 1)
    lpsrc_v[...] = k
    lpdst_v[...] = (g >> 7) * _TOTAL + 33 * k + ((32 * k * 4129) >> 19)
    lp_gather = pltpu.async_copy(lps_hbm.at[lpsrc_v], lpbuf, lpsem)

    def read_copy(i, b):
        return pltpu.make_async_copy(
            x_hbm.at[pl.ds(base + i * _CHUNK, _CHUNK)], bufs[b], rsems[b])

    def write_copy(i, b):
        return pltpu.make_async_copy(bufs[b], out_hbm.at[idx_v.at[i]],
                                     wsems[b])

    read_copy(0, 0).start()
    read_copy(1, 1).start()

    for c in range(_NCHUNK):
        for j in range(_CHUNK // 16):
            i = base + c * _CHUNK + j * 16 + lane
            r = i & (_INPUT_LEN - 1)
            idx_v[c, pl.ds(j * 16, 16)] = (
                i + ((i >> 12) << 7) + ((127 * r + 126) >> 12) + 1)

    lp_gather.wait()
    lp_scatter = pltpu.async_copy(lpbuf, out_hbm.at[lpdst_v], lpsem)

    def group(gi, carry):
        for b in range(_NBUF):
            i = _NBUF * gi + b
            read_copy(i, b).wait()
            write_copy(i, b).start()

            @pl.when(gi + 1 < _NCHUNK // _NBUF)
            def _():
                write_copy(i, b).wait()
                read_copy(i + _NBUF, b).start()
        return carry

    lax.fori_loop(0, _NCHUNK // _NBUF, group, 0)

    write_copy(_NCHUNK - _NBUF, 0).wait()
    write_copy(_NCHUNK - 1, 1).wait()
    lp_scatter.wait()


@jax.jit
def _run(x, lps):
    x2d = x.reshape(_XROWS, _DIM)
    mesh = plsc.VectorSubcoreMesh(
        core_axis_name="c", subcore_axis_name="s",
        num_cores=_NC, num_subcores=_NS)
    out = pl.kernel(
        _body,
        out_type=jax.ShapeDtypeStruct((_BATCH * _TOTAL, _DIM), jnp.float32),
        mesh=mesh,
        scratch_types=[
            pltpu.VMEM((_NCHUNK, _CHUNK), jnp.int32),
            pltpu.VMEM((_CHUNK, _DIM), jnp.float32),
            pltpu.VMEM((_CHUNK, _DIM), jnp.float32),
            pltpu.VMEM((16,), jnp.int32),
            pltpu.VMEM((16,), jnp.int32),
            pltpu.VMEM((_LP_PER_W, _DIM), jnp.float32),
            pltpu.SemaphoreType.DMA,
            pltpu.SemaphoreType.DMA,
            pltpu.SemaphoreType.DMA,
            pltpu.SemaphoreType.DMA,
            pltpu.SemaphoreType.DMA,
        ],
    )(x2d, lps)
    return out.reshape(_BATCH, _TOTAL, _DIM)


def kernel(x, lps):
    return _run(x, lps)
